# baseline (speedup 1.0000x reference)
import functools

import jax
import jax.numpy as jnp
from jax import lax
from jax.experimental import pallas as pl
from jax.experimental.pallas import tpu as pltpu
from jax.experimental.pallas import tpu_sc as plsc

N = 10000
E = 320000
D = 128
H = 128
NC = 2
NS = 16
NW = NC * NS
EPW = E // NW
CH = 80
NCH = EPW // CH
NP = 10240
RPS = NP // NS

_SC_MESH = plsc.VectorSubcoreMesh(core_axis_name="c", subcore_axis_name="s")


def _pq_body(mx, w1, w2, p_out, q_out):
    x = mx[...]
    p_out[...] = jnp.dot(x, w1[...], preferred_element_type=jnp.float32)
    q_out[...] = jnp.dot(x, w2[...], preferred_element_type=jnp.float32)


def _pq_call(mx, w1, w2):
    bn = 1000
    return pl.pallas_call(
        _pq_body,
        grid=(N // bn,),
        in_specs=[
            pl.BlockSpec((bn, D), lambda i: (i, 0)),
            pl.BlockSpec((D, H), lambda i: (0, 0)),
            pl.BlockSpec((D, H), lambda i: (0, 0)),
        ],
        out_specs=[
            pl.BlockSpec((bn, H), lambda i: (i, 0)),
            pl.BlockSpec((bn, H), lambda i: (i, 0)),
        ],
        out_shape=[
            jax.ShapeDtypeStruct((N, H), jnp.float32),
            jax.ShapeDtypeStruct((N, H), jnp.float32),
        ],
    )(mx, w1, w2)


@functools.partial(
    pl.kernel,
    out_type=jax.ShapeDtypeStruct((E, H), jnp.float32),
    mesh=_SC_MESH,
    scratch_types=[
        pltpu.VMEM((NCH, CH), jnp.int32),
        pltpu.VMEM((NCH, CH), jnp.int32),
        pltpu.VMEM((2, CH, H), jnp.float32),
        pltpu.VMEM((2, CH, H), jnp.float32),
        pltpu.SemaphoreType.DMA,
        pltpu.SemaphoreType.DMA,
        pltpu.SemaphoreType.DMA,
        pltpu.SemaphoreType.DMA,
    ],
)
def _gather_k(p_hbm, q_hbm, ei_hbm, ej_hbm, g_hbm, idxi, idxj,
              bufa, bufb, sa0, sb0, sa1, sb1):
    wid = lax.axis_index("s") * NC + lax.axis_index("c")
    base = wid * EPW
    pltpu.sync_copy(ei_hbm.at[wid], idxi)
    pltpu.sync_copy(ej_hbm.at[wid], idxj)
    sems = ((sa0, sb0), (sa1, sb1))

    def start(j, slot):
        sa, sb = sems[slot]
        pltpu.async_copy(p_hbm.at[idxi.at[j]], bufa.at[slot], sa)
        pltpu.async_copy(q_hbm.at[idxj.at[j]], bufb.at[slot], sb)

    def finish(j, slot):
        sa, sb = sems[slot]
        pltpu.make_async_copy(p_hbm.at[idxi.at[j]], bufa.at[slot], sa).wait()
        pltpu.make_async_copy(q_hbm.at[idxj.at[j]], bufb.at[slot], sb).wait()

        def row(r, c2):
            for k in range(H // 16):
                sl = pl.ds(k * 16, 16)
                bufa[slot, r, sl] = bufa[slot, r, sl] + bufb[slot, r, sl]
            return c2

        lax.fori_loop(0, CH, row, 0)
        pltpu.sync_copy(bufa.at[slot], g_hbm.at[pl.ds(base + j * CH, CH)])

    start(0, 0)

    def pair(jj, carry):
        j0 = 2 * jj
        start(j0 + 1, 1)
        finish(j0, 0)
        start(j0 + 2, 0)
        finish(j0 + 1, 1)
        return carry

    lax.fori_loop(0, NCH // 2, pair, 0)
    finish(NCH - 1, 0)


@functools.partial(
    pl.kernel,
    out_type=jax.ShapeDtypeStruct((NC * NP, H), jnp.float32),
    mesh=_SC_MESH,
    scratch_types=[
        pltpu.VMEM((NCH, CH), jnp.int32),
        pltpu.VMEM((CH, H), jnp.float32),
        pltpu.VMEM((CH, H), jnp.float32),
        pltpu.VMEM_SHARED((NP, H), jnp.float32),
        pltpu.SemaphoreType.DMA,
    ],
)
def _count_k(ei_hbm, cnt_out, idx, ones_v, stage, cnt_sh, sem):
    c = lax.axis_index("c")
    s = lax.axis_index("s")
    wid = s * NC + c
    pltpu.sync_copy(ei_hbm.at[wid], idx)

    def fill(r, carry):
        for k in range(H // 16):
            sl = pl.ds(k * 16, 16)
            stage[r, sl] = jnp.zeros((16,), jnp.float32)
            ones_v[r, sl] = jnp.ones((16,), jnp.float32)
        return carry

    lax.fori_loop(0, CH, fill, 0)
    for t in range(RPS // CH):
        pltpu.sync_copy(stage, cnt_sh.at[pl.ds(s * RPS + t * CH, CH)])
    plsc.subcore_barrier()

    def fire(j):
        pltpu.async_copy(ones_v, cnt_sh.at[idx.at[j]], sem, add=True)

    def drain(j):
        pltpu.make_async_copy(ones_v, cnt_sh.at[idx.at[j]], sem).wait()

    w = 4
    for j0 in range(w):
        fire(j0)

    def chunk(j, carry):
        fire(j + w)
        drain(j)
        return carry

    lax.fori_loop(0, NCH - w, chunk, 0)
    for t in range(w):
        drain(NCH - w + t)
    plsc.subcore_barrier()
    for t in range(RPS // CH):
        pltpu.sync_copy(cnt_sh.at[pl.ds(s * RPS + t * CH, CH)], stage)
        pltpu.sync_copy(stage, cnt_out.at[pl.ds(c * NP + s * RPS + t * CH, CH)])


def _edge_body(g, ef, w3, b_in, wres, bres, out):
    u = jnp.dot(ef[...], w3[...], preferred_element_type=jnp.float32)
    h0 = jnp.maximum(g[...] + u + b_in[...], 0.0)
    h1 = h0 + jnp.dot(h0.astype(jnp.bfloat16), wres[...],
                      preferred_element_type=jnp.float32)
    out[...] = jnp.maximum(h1 + bres[...], 0.0)


def _edge_call(g, ef, w3, b_in, wres, bres):
    be = 3200
    return pl.pallas_call(
        _edge_body,
        grid=(E // be,),
        in_specs=[
            pl.BlockSpec((be, H), lambda i: (i, 0)),
            pl.BlockSpec((be, D), lambda i: (i, 0)),
            pl.BlockSpec((D, H), lambda i: (0, 0)),
            pl.BlockSpec((1, H), lambda i: (0, 0)),
            pl.BlockSpec((H, H), lambda i: (0, 0)),
            pl.BlockSpec((1, H), lambda i: (0, 0)),
        ],
        out_specs=pl.BlockSpec((be, H), lambda i: (i, 0)),
        out_shape=jax.ShapeDtypeStruct((E, H), jnp.float32),
    )(g, ef, w3, b_in, wres, bres)


@functools.partial(
    pl.kernel,
    out_type=jax.ShapeDtypeStruct((NC * NP, H), jnp.float32),
    mesh=_SC_MESH,
    scratch_types=[
        pltpu.VMEM((NCH, CH), jnp.int32),
        pltpu.VMEM((2, CH, H), jnp.float32),
        pltpu.VMEM_SHARED((NP, H), jnp.float32),
        pltpu.SemaphoreType.DMA,
        pltpu.SemaphoreType.DMA,
    ],
)
def _scatter_k(h_hbm, ei_hbm, acc_out, idx, rows, acc_sh, s0, s1):
    c = lax.axis_index("c")
    s = lax.axis_index("s")
    wid = s * NC + c
    sems = (s0, s1)

    def fill(r, carry):
        for k in range(H // 16):
            rows[0, r, pl.ds(k * 16, 16)] = jnp.zeros((16,), jnp.float32)
        return carry

    lax.fori_loop(0, CH, fill, 0)
    for t in range(RPS // CH):
        pltpu.sync_copy(rows.at[0], acc_sh.at[pl.ds(s * RPS + t * CH, CH)])
    pltpu.sync_copy(ei_hbm.at[wid], idx)
    plsc.subcore_barrier()

    def start(j, slot):
        pltpu.async_copy(h_hbm.at[pl.ds(wid * EPW + j * CH, CH)],
                         rows.at[slot], sems[slot])

    def finish(j, slot):
        pltpu.make_async_copy(h_hbm.at[pl.ds(wid * EPW + j * CH, CH)],
                              rows.at[slot], sems[slot]).wait()
        pltpu.sync_copy(rows.at[slot], acc_sh.at[idx.at[j]], add=True)

    start(0, 0)

    def pair(jj, carry):
        j0 = 2 * jj
        start(j0 + 1, 1)
        finish(j0, 0)
        start(j0 + 2, 0)
        finish(j0 + 1, 1)
        return carry

    lax.fori_loop(0, NCH // 2, pair, 0)
    finish(NCH - 1, 0)
    plsc.subcore_barrier()
    for t in range(RPS // CH):
        sl = pl.ds(s * RPS + t * CH, CH)
        pltpu.sync_copy(acc_sh.at[sl], rows.at[0])
        pltpu.sync_copy(rows.at[0], acc_out.at[pl.ds(c * NP + s * RPS + t * CH, CH)])


def _node_body(mx, acc, cnt, gs, w_out, b_out, wn1, wn2, wn3, nb_in,
               wres, bres, wno, bno, out):
    s_sum = acc[0] + acc[1]
    cvec = cnt[0, :, 0:1] + cnt[1, :, 0:1]
    so = jnp.dot(s_sum, w_out[...], preferred_element_type=jnp.float32)
    agg = jnp.where(cvec > 0.0, so / jnp.maximum(cvec, 1.0) + b_out[...], 0.0)
    h = (jnp.dot(mx[...], wn1[...], preferred_element_type=jnp.float32)
         + jnp.dot(agg, wn2[...], preferred_element_type=jnp.float32)
         + gs[...] * wn3[...] + nb_in[...])
    h = jnp.maximum(h, 0.0)
    h = jnp.maximum(
        h + jnp.dot(h, wres[...], preferred_element_type=jnp.float32)
        + bres[...], 0.0)
    out[...] = jnp.dot(h, wno[...], preferred_element_type=jnp.float32) + bno[...]


def _node_call(mx, acc, cnt, gs, w_out, b_out, wn1, wn2, wn3, nb_in,
               wres, bres, wno, bno):
    bn = 1000
    full = lambda i: (0, 0)
    return pl.pallas_call(
        _node_body,
        grid=(N // bn,),
        in_specs=[
            pl.BlockSpec((bn, D), lambda i: (i, 0)),
            pl.BlockSpec((NC, bn, H), lambda i: (0, i, 0)),
            pl.BlockSpec((NC, bn, H), lambda i: (0, i, 0)),
            pl.BlockSpec((bn, 1), lambda i: (i, 0)),
            pl.BlockSpec((H, D), full),
            pl.BlockSpec((1, D), full),
            pl.BlockSpec((D, H), full),
            pl.BlockSpec((D, H), full),
            pl.BlockSpec((1, H), full),
            pl.BlockSpec((1, H), full),
            pl.BlockSpec((H, H), full),
            pl.BlockSpec((1, H), full),
            pl.BlockSpec((H, D), full),
            pl.BlockSpec((1, D), full),
        ],
        out_specs=pl.BlockSpec((bn, D), lambda i: (i, 0)),
        out_shape=jax.ShapeDtypeStruct((N, D), jnp.float32),
    )(mx, acc, cnt, gs, w_out, b_out, wn1, wn2, wn3, nb_in, wres, bres,
      wno, bno)


def kernel(meta_xs, edge_index, edge_feature, global_state,
           bWin, bbin, bWres, bbres, bWout, bbout,
           nWin, nbin, nWres, nbres, nWout, nbout):
    ei3 = edge_index[0].reshape(NW, NCH, CH)
    ej3 = edge_index[1].reshape(NW, NCH, CH)
    w1, w2, w3 = bWin[:D], bWin[D:2 * D], bWin[2 * D:]
    p, q = _pq_call(meta_xs, w1, w2)
    g = _gather_k(p, q, ei3, ej3)
    h = _edge_call(g, edge_feature.astype(jnp.bfloat16),
                   w3.astype(jnp.bfloat16), bbin.reshape(1, H),
                   bWres.astype(jnp.bfloat16), bbres.reshape(1, H))
    cnt_f = _count_k(ei3)
    acc = _scatter_k(h, ei3).reshape(NC, NP, H)
    cnt = cnt_f.reshape(NC, NP, H)
    wn1, wn2, wn3 = nWin[:D], nWin[D:2 * D], nWin[2 * D:2 * D + 1]
    return _node_call(
        meta_xs, acc, cnt, global_state.reshape(N, 1),
        bWout, bbout.reshape(1, D), wn1, wn2, wn3, nbin.reshape(1, H),
        nWres, nbres.reshape(1, H), nWout, nbout.reshape(1, D))


# speedup vs baseline: 1.0959x; 1.0959x over previous
import functools

import jax
import jax.numpy as jnp
from jax import lax
from jax.experimental import pallas as pl
from jax.experimental.pallas import tpu as pltpu
from jax.experimental.pallas import tpu_sc as plsc

N = 10000
E = 320000
D = 128
H = 128
NC = 2
NS = 16
NW = NC * NS
EPW = E // NW
CH = 80
NCH = EPW // CH
NP = 10240
RPS = NP // NS

_SC_MESH = plsc.VectorSubcoreMesh(core_axis_name="c", subcore_axis_name="s")


def _pq_body(mx, w1, w2, p_out, q_out):
    x = mx[...]
    p_out[...] = jnp.dot(x, w1[...], preferred_element_type=jnp.float32)
    q_out[...] = jnp.dot(x, w2[...], preferred_element_type=jnp.float32)


def _pq_call(mx, w1, w2):
    bn = 1000
    return pl.pallas_call(
        _pq_body,
        grid=(N // bn,),
        in_specs=[
            pl.BlockSpec((bn, D), lambda i: (i, 0)),
            pl.BlockSpec((D, H), lambda i: (0, 0)),
            pl.BlockSpec((D, H), lambda i: (0, 0)),
        ],
        out_specs=[
            pl.BlockSpec((bn, H), lambda i: (i, 0)),
            pl.BlockSpec((bn, H), lambda i: (i, 0)),
        ],
        out_shape=[
            jax.ShapeDtypeStruct((N, H), jnp.float32),
            jax.ShapeDtypeStruct((N, H), jnp.float32),
        ],
    )(mx, w1, w2)


@functools.partial(
    pl.kernel,
    out_type=jax.ShapeDtypeStruct((E, H), jnp.float32),
    mesh=_SC_MESH,
    scratch_types=[
        pltpu.VMEM((NCH, CH), jnp.int32),
        pltpu.VMEM((NCH, CH), jnp.int32),
        pltpu.VMEM((2, CH, H), jnp.float32),
        pltpu.VMEM((2, CH, H), jnp.float32),
        pltpu.SemaphoreType.DMA,
        pltpu.SemaphoreType.DMA,
        pltpu.SemaphoreType.DMA,
        pltpu.SemaphoreType.DMA,
    ],
)
def _gather_k(p_hbm, q_hbm, ei_hbm, ej_hbm, g_hbm, idxi, idxj,
              bufa, bufb, sa0, sb0, sa1, sb1):
    wid = lax.axis_index("s") * NC + lax.axis_index("c")
    base = wid * EPW
    pltpu.sync_copy(ei_hbm.at[wid], idxi)
    pltpu.sync_copy(ej_hbm.at[wid], idxj)
    sems = ((sa0, sb0), (sa1, sb1))

    def start(j, slot):
        sa, sb = sems[slot]
        pltpu.async_copy(p_hbm.at[idxi.at[j]], bufa.at[slot], sa)
        pltpu.async_copy(q_hbm.at[idxj.at[j]], bufb.at[slot], sb)

    def finish(j, slot):
        sa, sb = sems[slot]
        pltpu.make_async_copy(p_hbm.at[idxi.at[j]], bufa.at[slot], sa).wait()
        pltpu.make_async_copy(q_hbm.at[idxj.at[j]], bufb.at[slot], sb).wait()

        def row(r, c2):
            for k in range(H // 16):
                sl = pl.ds(k * 16, 16)
                bufa[slot, r, sl] = bufa[slot, r, sl] + bufb[slot, r, sl]
            return c2

        lax.fori_loop(0, CH, row, 0)
        pltpu.sync_copy(bufa.at[slot], g_hbm.at[pl.ds(base + j * CH, CH)])

    start(0, 0)

    def pair(jj, carry):
        j0 = 2 * jj
        start(j0 + 1, 1)
        finish(j0, 0)
        start(j0 + 2, 0)
        finish(j0 + 1, 1)
        return carry

    lax.fori_loop(0, NCH // 2, pair, 0)
    finish(NCH - 1, 0)


@functools.partial(
    pl.kernel,
    out_type=jax.ShapeDtypeStruct((NC * NP, H), jnp.float32),
    mesh=_SC_MESH,
    scratch_types=[
        pltpu.VMEM((NCH, CH), jnp.int32),
        pltpu.VMEM((CH, H), jnp.float32),
        pltpu.VMEM((CH, H), jnp.float32),
        pltpu.VMEM_SHARED((NP, H), jnp.float32),
        pltpu.SemaphoreType.DMA,
    ],
)
def _count_k(ei_hbm, cnt_out, idx, ones_v, stage, cnt_sh, sem):
    c = lax.axis_index("c")
    s = lax.axis_index("s")
    wid = s * NC + c
    pltpu.sync_copy(ei_hbm.at[wid], idx)

    def fill(r, carry):
        for k in range(H // 16):
            sl = pl.ds(k * 16, 16)
            stage[r, sl] = jnp.zeros((16,), jnp.float32)
            ones_v[r, sl] = jnp.ones((16,), jnp.float32)
        return carry

    lax.fori_loop(0, CH, fill, 0)
    for t in range(RPS // CH):
        pltpu.sync_copy(stage, cnt_sh.at[pl.ds(s * RPS + t * CH, CH)])
    plsc.subcore_barrier()

    def fire(j):
        pltpu.async_copy(ones_v, cnt_sh.at[idx.at[j]], sem, add=True)

    def drain(j):
        pltpu.make_async_copy(ones_v, cnt_sh.at[idx.at[j]], sem).wait()

    w = 4
    for j0 in range(w):
        fire(j0)

    def chunk(j, carry):
        fire(j + w)
        drain(j)
        return carry

    lax.fori_loop(0, NCH - w, chunk, 0)
    for t in range(w):
        drain(NCH - w + t)
    plsc.subcore_barrier()
    for t in range(RPS // CH):
        pltpu.sync_copy(cnt_sh.at[pl.ds(s * RPS + t * CH, CH)], stage)
        pltpu.sync_copy(stage, cnt_out.at[pl.ds(c * NP + s * RPS + t * CH, CH)])


def _edge_body(g, ef, w3, b_in, wres, bres, out):
    u = jnp.dot(ef[...].astype(jnp.bfloat16), w3[...].astype(jnp.bfloat16),
                preferred_element_type=jnp.float32)
    h0 = jnp.maximum(g[...] + u + b_in[...], 0.0)
    h1 = h0 + jnp.dot(h0.astype(jnp.bfloat16),
                      wres[...].astype(jnp.bfloat16),
                      preferred_element_type=jnp.float32)
    out[...] = jnp.maximum(h1 + bres[...], 0.0)


def _edge_call(g, ef, w3, b_in, wres, bres):
    be = 3200
    return pl.pallas_call(
        _edge_body,
        grid=(E // be,),
        in_specs=[
            pl.BlockSpec((be, H), lambda i: (i, 0)),
            pl.BlockSpec((be, D), lambda i: (i, 0)),
            pl.BlockSpec((D, H), lambda i: (0, 0)),
            pl.BlockSpec((1, H), lambda i: (0, 0)),
            pl.BlockSpec((H, H), lambda i: (0, 0)),
            pl.BlockSpec((1, H), lambda i: (0, 0)),
        ],
        out_specs=pl.BlockSpec((be, H), lambda i: (i, 0)),
        out_shape=jax.ShapeDtypeStruct((E, H), jnp.float32),
    )(g, ef, w3, b_in, wres, bres)


@functools.partial(
    pl.kernel,
    out_type=jax.ShapeDtypeStruct((NC * NP, H), jnp.float32),
    mesh=_SC_MESH,
    scratch_types=[
        pltpu.VMEM((NCH, CH), jnp.int32),
        pltpu.VMEM((2, CH, H), jnp.float32),
        pltpu.VMEM_SHARED((NP, H), jnp.float32),
        pltpu.SemaphoreType.DMA,
        pltpu.SemaphoreType.DMA,
    ],
)
def _scatter_k(h_hbm, ei_hbm, acc_out, idx, rows, acc_sh, s0, s1):
    c = lax.axis_index("c")
    s = lax.axis_index("s")
    wid = s * NC + c
    sems = (s0, s1)

    def fill(r, carry):
        for k in range(H // 16):
            rows[0, r, pl.ds(k * 16, 16)] = jnp.zeros((16,), jnp.float32)
        return carry

    lax.fori_loop(0, CH, fill, 0)
    for t in range(RPS // CH):
        pltpu.sync_copy(rows.at[0], acc_sh.at[pl.ds(s * RPS + t * CH, CH)])
    pltpu.sync_copy(ei_hbm.at[wid], idx)
    plsc.subcore_barrier()

    def start(j, slot):
        pltpu.async_copy(h_hbm.at[pl.ds(wid * EPW + j * CH, CH)],
                         rows.at[slot], sems[slot])

    def finish(j, slot):
        pltpu.make_async_copy(h_hbm.at[pl.ds(wid * EPW + j * CH, CH)],
                              rows.at[slot], sems[slot]).wait()
        pltpu.sync_copy(rows.at[slot], acc_sh.at[idx.at[j]], add=True)

    start(0, 0)

    def pair(jj, carry):
        j0 = 2 * jj
        start(j0 + 1, 1)
        finish(j0, 0)
        start(j0 + 2, 0)
        finish(j0 + 1, 1)
        return carry

    lax.fori_loop(0, NCH // 2, pair, 0)
    finish(NCH - 1, 0)
    plsc.subcore_barrier()
    for t in range(RPS // CH):
        sl = pl.ds(s * RPS + t * CH, CH)
        pltpu.sync_copy(acc_sh.at[sl], rows.at[0])
        pltpu.sync_copy(rows.at[0], acc_out.at[pl.ds(c * NP + s * RPS + t * CH, CH)])


def _node_body(mx, acc, cnt, gs, w_out, b_out, wn1, wn2, wn3, nb_in,
               wres, bres, wno, bno, out):
    s_sum = acc[0] + acc[1]
    cvec = cnt[0, :, 0:1] + cnt[1, :, 0:1]
    so = jnp.dot(s_sum, w_out[...], preferred_element_type=jnp.float32)
    agg = jnp.where(cvec > 0.0, so / jnp.maximum(cvec, 1.0) + b_out[...], 0.0)
    h = (jnp.dot(mx[...], wn1[...], preferred_element_type=jnp.float32)
         + jnp.dot(agg, wn2[...], preferred_element_type=jnp.float32)
         + gs[...] * wn3[...] + nb_in[...])
    h = jnp.maximum(h, 0.0)
    h = jnp.maximum(
        h + jnp.dot(h, wres[...], preferred_element_type=jnp.float32)
        + bres[...], 0.0)
    out[...] = jnp.dot(h, wno[...], preferred_element_type=jnp.float32) + bno[...]


def _node_call(mx, acc, cnt, gs, w_out, b_out, wn1, wn2, wn3, nb_in,
               wres, bres, wno, bno):
    bn = 1000
    full = lambda i: (0, 0)
    return pl.pallas_call(
        _node_body,
        grid=(N // bn,),
        in_specs=[
            pl.BlockSpec((bn, D), lambda i: (i, 0)),
            pl.BlockSpec((NC, bn, H), lambda i: (0, i, 0)),
            pl.BlockSpec((NC, bn, H), lambda i: (0, i, 0)),
            pl.BlockSpec((bn, 1), lambda i: (i, 0)),
            pl.BlockSpec((H, D), full),
            pl.BlockSpec((1, D), full),
            pl.BlockSpec((D, H), full),
            pl.BlockSpec((D, H), full),
            pl.BlockSpec((1, H), full),
            pl.BlockSpec((1, H), full),
            pl.BlockSpec((H, H), full),
            pl.BlockSpec((1, H), full),
            pl.BlockSpec((H, D), full),
            pl.BlockSpec((1, D), full),
        ],
        out_specs=pl.BlockSpec((bn, D), lambda i: (i, 0)),
        out_shape=jax.ShapeDtypeStruct((N, D), jnp.float32),
    )(mx, acc, cnt, gs, w_out, b_out, wn1, wn2, wn3, nb_in, wres, bres,
      wno, bno)


def kernel(meta_xs, edge_index, edge_feature, global_state,
           bWin, bbin, bWres, bbres, bWout, bbout,
           nWin, nbin, nWres, nbres, nWout, nbout):
    ei3 = edge_index[0].reshape(NW, NCH, CH)
    ej3 = edge_index[1].reshape(NW, NCH, CH)
    w1, w2, w3 = bWin[:D], bWin[D:2 * D], bWin[2 * D:]
    p, q = _pq_call(meta_xs, w1, w2)
    g = _gather_k(p, q, ei3, ej3)
    cnt_f = _count_k(ei3)
    h = _edge_call(g, edge_feature, w3, bbin.reshape(1, H),
                   bWres, bbres.reshape(1, H))
    acc = _scatter_k(h, ei3).reshape(NC, NP, H)
    cnt = cnt_f.reshape(NC, NP, H)
    wn1, wn2, wn3 = nWin[:D], nWin[D:2 * D], nWin[2 * D:2 * D + 1]
    return _node_call(
        meta_xs, acc, cnt, global_state.reshape(N, 1),
        bWout, bbout.reshape(1, D), wn1, wn2, wn3, nbin.reshape(1, H),
        nWres, nbres.reshape(1, H), nWout, nbout.reshape(1, D))


# speedup vs baseline: 1.1100x; 1.0129x over previous
import functools

import jax
import jax.numpy as jnp
from jax import lax
from jax.experimental import pallas as pl
from jax.experimental.pallas import tpu as pltpu
from jax.experimental.pallas import tpu_sc as plsc

N = 10000
E = 320000
D = 128
H = 128
NC = 2
NS = 16
NW = NC * NS
EPW = E // NW
CH = 80
NCH = EPW // CH
NP = 10240
RPS = NP // NS

_SC_MESH = plsc.VectorSubcoreMesh(core_axis_name="c", subcore_axis_name="s")


def _pq_body(mx, w1, w2, p_out, q_out):
    x = mx[...]
    p_out[...] = jnp.dot(x, w1[...], preferred_element_type=jnp.float32)
    q_out[...] = jnp.dot(x, w2[...], preferred_element_type=jnp.float32)


def _pq_call(mx, w1, w2):
    bn = 1000
    return pl.pallas_call(
        _pq_body,
        grid=(N // bn,),
        in_specs=[
            pl.BlockSpec((bn, D), lambda i: (i, 0)),
            pl.BlockSpec((D, H), lambda i: (0, 0)),
            pl.BlockSpec((D, H), lambda i: (0, 0)),
        ],
        out_specs=[
            pl.BlockSpec((bn, H), lambda i: (i, 0)),
            pl.BlockSpec((bn, H), lambda i: (i, 0)),
        ],
        out_shape=[
            jax.ShapeDtypeStruct((N, H), jnp.float32),
            jax.ShapeDtypeStruct((N, H), jnp.float32),
        ],
    )(mx, w1, w2)


@functools.partial(
    pl.kernel,
    out_type=jax.ShapeDtypeStruct((E, H), jnp.float32),
    mesh=_SC_MESH,
    scratch_types=[
        pltpu.VMEM((NCH, CH), jnp.int32),
        pltpu.VMEM((NCH, CH), jnp.int32),
        pltpu.VMEM((2, CH, H), jnp.float32),
        pltpu.VMEM((2, CH, H), jnp.float32),
        pltpu.VMEM((2, CH, H), jnp.float32),
        pltpu.SemaphoreType.DMA,
        pltpu.SemaphoreType.DMA,
        pltpu.SemaphoreType.DMA,
        pltpu.SemaphoreType.DMA,
        pltpu.SemaphoreType.DMA,
        pltpu.SemaphoreType.DMA,
    ],
)
def _gather_k(p_hbm, q_hbm, ei_hbm, ej_hbm, g_hbm, idxi, idxj,
              bufa, bufb, bufo, sa0, sb0, sa1, sb1, sw0, sw1):
    wid = lax.axis_index("s") * NC + lax.axis_index("c")
    base = wid * EPW
    pltpu.sync_copy(ei_hbm.at[wid], idxi)
    pltpu.sync_copy(ej_hbm.at[wid], idxj)
    sems = ((sa0, sb0), (sa1, sb1))
    wsems = (sw0, sw1)

    def start(j, slot):
        sa, sb = sems[slot]
        pltpu.async_copy(p_hbm.at[idxi.at[j]], bufa.at[slot], sa)
        pltpu.async_copy(q_hbm.at[idxj.at[j]], bufb.at[slot], sb)

    def work(j, slot):
        sa, sb = sems[slot]
        pltpu.make_async_copy(p_hbm.at[idxi.at[j]], bufa.at[slot], sa).wait()
        pltpu.make_async_copy(q_hbm.at[idxj.at[j]], bufb.at[slot], sb).wait()

        def row(r, c2):
            for k in range(H // 16):
                sl = pl.ds(k * 16, 16)
                bufo[slot, r, sl] = bufa[slot, r, sl] + bufb[slot, r, sl]
            return c2

        lax.fori_loop(0, CH, row, 0)
        pltpu.async_copy(bufo.at[slot], g_hbm.at[pl.ds(base + j * CH, CH)],
                         wsems[slot])

    def drainw(j, slot):
        pltpu.make_async_copy(bufo.at[slot],
                              g_hbm.at[pl.ds(base + j * CH, CH)],
                              wsems[slot]).wait()

    start(0, 0)
    start(1, 1)
    work(0, 0)
    start(2, 0)
    work(1, 1)

    def pair(jj, carry):
        j0 = 2 * jj
        start(j0 + 1, 1)
        drainw(j0 - 2, 0)
        work(j0, 0)
        start(j0 + 2, 0)
        drainw(j0 - 1, 1)
        work(j0 + 1, 1)
        return carry

    lax.fori_loop(1, NCH // 2, pair, 0)
    drainw(NCH - 3, 0)
    work(NCH - 1, 0)
    drainw(NCH - 2, 1)
    drainw(NCH - 1, 0)


@functools.partial(
    pl.kernel,
    out_type=jax.ShapeDtypeStruct((NC * NP, H), jnp.float32),
    mesh=_SC_MESH,
    scratch_types=[
        pltpu.VMEM((NCH, CH), jnp.int32),
        pltpu.VMEM((CH, H), jnp.float32),
        pltpu.VMEM((CH, H), jnp.float32),
        pltpu.VMEM_SHARED((NP, H), jnp.float32),
        pltpu.SemaphoreType.DMA,
    ],
)
def _count_k(ei_hbm, cnt_out, idx, ones_v, stage, cnt_sh, sem):
    c = lax.axis_index("c")
    s = lax.axis_index("s")
    wid = s * NC + c
    pltpu.sync_copy(ei_hbm.at[wid], idx)

    def fill(r, carry):
        for k in range(H // 16):
            sl = pl.ds(k * 16, 16)
            stage[r, sl] = jnp.zeros((16,), jnp.float32)
            ones_v[r, sl] = jnp.ones((16,), jnp.float32)
        return carry

    lax.fori_loop(0, CH, fill, 0)
    for t in range(RPS // CH):
        pltpu.sync_copy(stage, cnt_sh.at[pl.ds(s * RPS + t * CH, CH)])
    plsc.subcore_barrier()

    def fire(j):
        pltpu.async_copy(ones_v, cnt_sh.at[idx.at[j]], sem, add=True)

    def drain(j):
        pltpu.make_async_copy(ones_v, cnt_sh.at[idx.at[j]], sem).wait()

    w = 4
    for j0 in range(w):
        fire(j0)

    def chunk(j, carry):
        fire(j + w)
        drain(j)
        return carry

    lax.fori_loop(0, NCH - w, chunk, 0)
    for t in range(w):
        drain(NCH - w + t)
    plsc.subcore_barrier()
    for t in range(RPS // CH):
        pltpu.sync_copy(cnt_sh.at[pl.ds(s * RPS + t * CH, CH)], stage)
        pltpu.sync_copy(stage, cnt_out.at[pl.ds(c * NP + s * RPS + t * CH, CH)])


def _edge_body(g, ef, w3, b_in, wres, bres, out):
    u = jnp.dot(ef[...].astype(jnp.bfloat16), w3[...].astype(jnp.bfloat16),
                preferred_element_type=jnp.float32)
    h0 = jnp.maximum(g[...] + u + b_in[...], 0.0)
    h1 = h0 + jnp.dot(h0.astype(jnp.bfloat16),
                      wres[...].astype(jnp.bfloat16),
                      preferred_element_type=jnp.float32)
    out[...] = jnp.maximum(h1 + bres[...], 0.0)


def _edge_call(g, ef, w3, b_in, wres, bres):
    be = 3200
    return pl.pallas_call(
        _edge_body,
        grid=(E // be,),
        in_specs=[
            pl.BlockSpec((be, H), lambda i: (i, 0)),
            pl.BlockSpec((be, D), lambda i: (i, 0)),
            pl.BlockSpec((D, H), lambda i: (0, 0)),
            pl.BlockSpec((1, H), lambda i: (0, 0)),
            pl.BlockSpec((H, H), lambda i: (0, 0)),
            pl.BlockSpec((1, H), lambda i: (0, 0)),
        ],
        out_specs=pl.BlockSpec((be, H), lambda i: (i, 0)),
        out_shape=jax.ShapeDtypeStruct((E, H), jnp.float32),
    )(g, ef, w3, b_in, wres, bres)


@functools.partial(
    pl.kernel,
    out_type=jax.ShapeDtypeStruct((NC * NP, H), jnp.float32),
    mesh=_SC_MESH,
    scratch_types=[
        pltpu.VMEM((NCH, CH), jnp.int32),
        pltpu.VMEM((2, CH, H), jnp.float32),
        pltpu.VMEM_SHARED((NP, H), jnp.float32),
        pltpu.SemaphoreType.DMA,
        pltpu.SemaphoreType.DMA,
    ],
)
def _scatter_k(h_hbm, ei_hbm, acc_out, idx, rows, acc_sh, s0, s1):
    c = lax.axis_index("c")
    s = lax.axis_index("s")
    wid = s * NC + c
    sems = (s0, s1)

    def fill(r, carry):
        for k in range(H // 16):
            rows[0, r, pl.ds(k * 16, 16)] = jnp.zeros((16,), jnp.float32)
        return carry

    lax.fori_loop(0, CH, fill, 0)
    for t in range(RPS // CH):
        pltpu.sync_copy(rows.at[0], acc_sh.at[pl.ds(s * RPS + t * CH, CH)])
    pltpu.sync_copy(ei_hbm.at[wid], idx)
    plsc.subcore_barrier()

    def start(j, slot):
        pltpu.async_copy(h_hbm.at[pl.ds(wid * EPW + j * CH, CH)],
                         rows.at[slot], sems[slot])

    def finish(j, slot):
        pltpu.make_async_copy(h_hbm.at[pl.ds(wid * EPW + j * CH, CH)],
                              rows.at[slot], sems[slot]).wait()
        pltpu.sync_copy(rows.at[slot], acc_sh.at[idx.at[j]], add=True)

    start(0, 0)

    def pair(jj, carry):
        j0 = 2 * jj
        start(j0 + 1, 1)
        finish(j0, 0)
        start(j0 + 2, 0)
        finish(j0 + 1, 1)
        return carry

    lax.fori_loop(0, NCH // 2, pair, 0)
    finish(NCH - 1, 0)
    plsc.subcore_barrier()
    for t in range(RPS // CH):
        sl = pl.ds(s * RPS + t * CH, CH)
        pltpu.sync_copy(acc_sh.at[sl], rows.at[0])
        pltpu.sync_copy(rows.at[0], acc_out.at[pl.ds(c * NP + s * RPS + t * CH, CH)])


def _node_body(mx, acc, cnt, gs, w_out, b_out, wn1, wn2, wn3, nb_in,
               wres, bres, wno, bno, out):
    s_sum = acc[0] + acc[1]
    cvec = cnt[0, :, 0:1] + cnt[1, :, 0:1]
    so = jnp.dot(s_sum, w_out[...], preferred_element_type=jnp.float32)
    agg = jnp.where(cvec > 0.0, so / jnp.maximum(cvec, 1.0) + b_out[...], 0.0)
    h = (jnp.dot(mx[...], wn1[...], preferred_element_type=jnp.float32)
         + jnp.dot(agg, wn2[...], preferred_element_type=jnp.float32)
         + gs[...] * wn3[...] + nb_in[...])
    h = jnp.maximum(h, 0.0)
    h = jnp.maximum(
        h + jnp.dot(h, wres[...], preferred_element_type=jnp.float32)
        + bres[...], 0.0)
    out[...] = jnp.dot(h, wno[...], preferred_element_type=jnp.float32) + bno[...]


def _node_call(mx, acc, cnt, gs, w_out, b_out, wn1, wn2, wn3, nb_in,
               wres, bres, wno, bno):
    bn = 1000
    full = lambda i: (0, 0)
    return pl.pallas_call(
        _node_body,
        grid=(N // bn,),
        in_specs=[
            pl.BlockSpec((bn, D), lambda i: (i, 0)),
            pl.BlockSpec((NC, bn, H), lambda i: (0, i, 0)),
            pl.BlockSpec((NC, bn, H), lambda i: (0, i, 0)),
            pl.BlockSpec((bn, 1), lambda i: (i, 0)),
            pl.BlockSpec((H, D), full),
            pl.BlockSpec((1, D), full),
            pl.BlockSpec((D, H), full),
            pl.BlockSpec((D, H), full),
            pl.BlockSpec((1, H), full),
            pl.BlockSpec((1, H), full),
            pl.BlockSpec((H, H), full),
            pl.BlockSpec((1, H), full),
            pl.BlockSpec((H, D), full),
            pl.BlockSpec((1, D), full),
        ],
        out_specs=pl.BlockSpec((bn, D), lambda i: (i, 0)),
        out_shape=jax.ShapeDtypeStruct((N, D), jnp.float32),
    )(mx, acc, cnt, gs, w_out, b_out, wn1, wn2, wn3, nb_in, wres, bres,
      wno, bno)


def kernel(meta_xs, edge_index, edge_feature, global_state,
           bWin, bbin, bWres, bbres, bWout, bbout,
           nWin, nbin, nWres, nbres, nWout, nbout):
    ei3 = edge_index[0].reshape(NW, NCH, CH)
    ej3 = edge_index[1].reshape(NW, NCH, CH)
    w1, w2, w3 = bWin[:D], bWin[D:2 * D], bWin[2 * D:]
    p, q = _pq_call(meta_xs, w1, w2)
    g = _gather_k(p, q, ei3, ej3)
    cnt_f = _count_k(ei3)
    h = _edge_call(g, edge_feature, w3, bbin.reshape(1, H),
                   bWres, bbres.reshape(1, H))
    acc = _scatter_k(h, ei3).reshape(NC, NP, H)
    cnt = cnt_f.reshape(NC, NP, H)
    wn1, wn2, wn3 = nWin[:D], nWin[D:2 * D], nWin[2 * D:2 * D + 1]
    return _node_call(
        meta_xs, acc, cnt, global_state.reshape(N, 1),
        bWout, bbout.reshape(1, D), wn1, wn2, wn3, nbin.reshape(1, H),
        nWres, nbres.reshape(1, H), nWout, nbout.reshape(1, D))


# speedup vs baseline: 1.1319x; 1.0197x over previous
import functools

import jax
import jax.numpy as jnp
from jax import lax
from jax.experimental import pallas as pl
from jax.experimental.pallas import tpu as pltpu
from jax.experimental.pallas import tpu_sc as plsc

N = 10000
E = 320000
D = 128
H = 128
NC = 2
NS = 16
NW = NC * NS
EPW = E // NW
CH = 80
NCH = EPW // CH
NP = 10240
RPS = NP // NS

_SC_MESH = plsc.VectorSubcoreMesh(core_axis_name="c", subcore_axis_name="s")


def _pq_body(mx, w1, w2, p_out, q_out):
    x = mx[...]
    p_out[...] = jnp.dot(x, w1[...], preferred_element_type=jnp.float32)
    q_out[...] = jnp.dot(x, w2[...], preferred_element_type=jnp.float32)


def _pq_call(mx, w1, w2):
    bn = 1000
    return pl.pallas_call(
        _pq_body,
        grid=(N // bn,),
        in_specs=[
            pl.BlockSpec((bn, D), lambda i: (i, 0)),
            pl.BlockSpec((D, H), lambda i: (0, 0)),
            pl.BlockSpec((D, H), lambda i: (0, 0)),
        ],
        out_specs=[
            pl.BlockSpec((bn, H), lambda i: (i, 0)),
            pl.BlockSpec((bn, H), lambda i: (i, 0)),
        ],
        out_shape=[
            jax.ShapeDtypeStruct((N, H), jnp.float32),
            jax.ShapeDtypeStruct((N, H), jnp.float32),
        ],
    )(mx, w1, w2)


@functools.partial(
    pl.kernel,
    out_type=jax.ShapeDtypeStruct((E, H), jnp.float32),
    mesh=_SC_MESH,
    scratch_types=[
        pltpu.VMEM((NCH, CH), jnp.int32),
        pltpu.VMEM((NCH, CH), jnp.int32),
        pltpu.VMEM((2, CH, H), jnp.float32),
        pltpu.VMEM((2, CH, H), jnp.float32),
        pltpu.VMEM((2, CH, H), jnp.float32),
        pltpu.SemaphoreType.DMA,
        pltpu.SemaphoreType.DMA,
        pltpu.SemaphoreType.DMA,
        pltpu.SemaphoreType.DMA,
        pltpu.SemaphoreType.DMA,
        pltpu.SemaphoreType.DMA,
    ],
)
def _gather_k(p_hbm, q_hbm, ei_hbm, ej_hbm, g_hbm, idxi, idxj,
              bufa, bufb, bufo, sa0, sb0, sa1, sb1, sw0, sw1):
    wid = lax.axis_index("s") * NC + lax.axis_index("c")
    base = wid * EPW
    pltpu.sync_copy(ei_hbm.at[wid], idxi)
    pltpu.sync_copy(ej_hbm.at[wid], idxj)
    sems = ((sa0, sb0), (sa1, sb1))
    wsems = (sw0, sw1)

    def start(j, slot):
        sa, sb = sems[slot]
        pltpu.async_copy(p_hbm.at[idxi.at[j]], bufa.at[slot], sa)
        pltpu.async_copy(q_hbm.at[idxj.at[j]], bufb.at[slot], sb)

    def work(j, slot):
        sa, sb = sems[slot]
        pltpu.make_async_copy(p_hbm.at[idxi.at[j]], bufa.at[slot], sa).wait()
        pltpu.make_async_copy(q_hbm.at[idxj.at[j]], bufb.at[slot], sb).wait()

        def row(r, c2):
            for k in range(H // 16):
                sl = pl.ds(k * 16, 16)
                bufo[slot, r, sl] = bufa[slot, r, sl] + bufb[slot, r, sl]
            return c2

        lax.fori_loop(0, CH, row, 0)
        pltpu.async_copy(bufo.at[slot], g_hbm.at[pl.ds(base + j * CH, CH)],
                         wsems[slot])

    def drainw(j, slot):
        pltpu.make_async_copy(bufo.at[slot],
                              g_hbm.at[pl.ds(base + j * CH, CH)],
                              wsems[slot]).wait()

    start(0, 0)
    start(1, 1)
    work(0, 0)
    start(2, 0)
    work(1, 1)

    def pair(jj, carry):
        j0 = 2 * jj
        start(j0 + 1, 1)
        drainw(j0 - 2, 0)
        work(j0, 0)
        start(j0 + 2, 0)
        drainw(j0 - 1, 1)
        work(j0 + 1, 1)
        return carry

    lax.fori_loop(1, NCH // 2, pair, 0)
    drainw(NCH - 3, 0)
    work(NCH - 1, 0)
    drainw(NCH - 2, 1)
    drainw(NCH - 1, 0)


@functools.partial(
    pl.kernel,
    out_type=jax.ShapeDtypeStruct((NC * NP, H), jnp.float32),
    mesh=_SC_MESH,
    scratch_types=[
        pltpu.VMEM((NCH, CH), jnp.int32),
        pltpu.VMEM((CH, H), jnp.float32),
        pltpu.VMEM((CH, H), jnp.float32),
        pltpu.VMEM_SHARED((NP, H), jnp.float32),
        pltpu.SemaphoreType.DMA,
    ],
)
def _count_k(ei_hbm, cnt_out, idx, ones_v, stage, cnt_sh, sem):
    c = lax.axis_index("c")
    s = lax.axis_index("s")
    wid = s * NC + c
    pltpu.sync_copy(ei_hbm.at[wid], idx)

    def fill(r, carry):
        for k in range(H // 16):
            sl = pl.ds(k * 16, 16)
            stage[r, sl] = jnp.zeros((16,), jnp.float32)
            ones_v[r, sl] = jnp.ones((16,), jnp.float32)
        return carry

    lax.fori_loop(0, CH, fill, 0)
    for t in range(RPS // CH):
        pltpu.sync_copy(stage, cnt_sh.at[pl.ds(s * RPS + t * CH, CH)])
    plsc.subcore_barrier()

    def fire(j):
        pltpu.async_copy(ones_v, cnt_sh.at[idx.at[j]], sem, add=True)

    def drain(j):
        pltpu.make_async_copy(ones_v, cnt_sh.at[idx.at[j]], sem).wait()

    w = 4
    for j0 in range(w):
        fire(j0)

    def chunk(j, carry):
        fire(j + w)
        drain(j)
        return carry

    lax.fori_loop(0, NCH - w, chunk, 0)
    for t in range(w):
        drain(NCH - w + t)
    plsc.subcore_barrier()
    for t in range(RPS // CH):
        pltpu.sync_copy(cnt_sh.at[pl.ds(s * RPS + t * CH, CH)], stage)
        pltpu.sync_copy(stage, cnt_out.at[pl.ds(c * NP + s * RPS + t * CH, CH)])


def _edge_body(g, ef, w3, b_in, wres, bres, out):
    u = jnp.dot(ef[...].astype(jnp.bfloat16), w3[...].astype(jnp.bfloat16),
                preferred_element_type=jnp.float32)
    h0 = jnp.maximum(g[...] + u + b_in[...], 0.0)
    h1 = h0 + jnp.dot(h0.astype(jnp.bfloat16),
                      wres[...].astype(jnp.bfloat16),
                      preferred_element_type=jnp.float32)
    out[...] = jnp.maximum(h1 + bres[...], 0.0)


def _edge_call(g, ef, w3, b_in, wres, bres):
    be = 3200
    return pl.pallas_call(
        _edge_body,
        grid=(E // be,),
        in_specs=[
            pl.BlockSpec((be, H), lambda i: (i, 0)),
            pl.BlockSpec((be, D), lambda i: (i, 0)),
            pl.BlockSpec((D, H), lambda i: (0, 0)),
            pl.BlockSpec((1, H), lambda i: (0, 0)),
            pl.BlockSpec((H, H), lambda i: (0, 0)),
            pl.BlockSpec((1, H), lambda i: (0, 0)),
        ],
        out_specs=pl.BlockSpec((be, H), lambda i: (i, 0)),
        out_shape=jax.ShapeDtypeStruct((E, H), jnp.float32),
    )(g, ef, w3, b_in, wres, bres)


@functools.partial(
    pl.kernel,
    out_type=jax.ShapeDtypeStruct((NC * NP, H), jnp.float32),
    mesh=_SC_MESH,
    scratch_types=[
        pltpu.VMEM((NCH, CH), jnp.int32),
        pltpu.VMEM((3, CH, H), jnp.float32),
        pltpu.VMEM_SHARED((NP, H), jnp.float32),
        pltpu.SemaphoreType.DMA,
        pltpu.SemaphoreType.DMA,
        pltpu.SemaphoreType.DMA,
        pltpu.SemaphoreType.DMA,
        pltpu.SemaphoreType.DMA,
        pltpu.SemaphoreType.DMA,
    ],
)
def _scatter_k(h_hbm, ei_hbm, acc_out, idx, rows, acc_sh,
               r0, r1, r2, a0, a1, a2):
    c = lax.axis_index("c")
    s = lax.axis_index("s")
    wid = s * NC + c
    rsem = (r0, r1, r2)
    asem = (a0, a1, a2)

    def fill(r, carry):
        for k in range(H // 16):
            rows[0, r, pl.ds(k * 16, 16)] = jnp.zeros((16,), jnp.float32)
        return carry

    lax.fori_loop(0, CH, fill, 0)
    for t in range(RPS // CH):
        pltpu.sync_copy(rows.at[0], acc_sh.at[pl.ds(s * RPS + t * CH, CH)])
    pltpu.sync_copy(ei_hbm.at[wid], idx)
    plsc.subcore_barrier()

    def start_r(j, slot):
        pltpu.async_copy(h_hbm.at[pl.ds(wid * EPW + j * CH, CH)],
                         rows.at[slot], rsem[slot])

    def drain_r(j, slot):
        pltpu.make_async_copy(h_hbm.at[pl.ds(wid * EPW + j * CH, CH)],
                              rows.at[slot], rsem[slot]).wait()

    def fire_a(j, slot):
        pltpu.async_copy(rows.at[slot], acc_sh.at[idx.at[j]], asem[slot],
                         add=True)

    def drain_a(j, slot):
        pltpu.make_async_copy(rows.at[slot], acc_sh.at[idx.at[j]],
                              asem[slot]).wait()

    start_r(0, 0)
    start_r(1, 1)
    start_r(2, 2)
    drain_r(0, 0)
    fire_a(0, 0)
    drain_r(1, 1)
    fire_a(1, 1)

    def triple(t, carry):
        j = 3 * t
        drain_a(j, 0)
        start_r(j + 3, 0)
        drain_r(j + 2, 2)
        fire_a(j + 2, 2)
        drain_a(j + 1, 1)
        start_r(j + 4, 1)
        drain_r(j + 3, 0)
        fire_a(j + 3, 0)
        drain_a(j + 2, 2)
        start_r(jnp.minimum(j + 5, NCH - 1), 2)
        drain_r(j + 4, 1)
        fire_a(j + 4, 1)
        return carry

    lax.fori_loop(0, (NCH - 2) // 3, triple, 0)
    drain_a(NCH - 2, 0)
    drain_a(NCH - 1, 1)
    drain_r(NCH - 1, 2)
    plsc.subcore_barrier()
    for t in range(RPS // CH):
        sl = pl.ds(s * RPS + t * CH, CH)
        pltpu.sync_copy(acc_sh.at[sl], rows.at[0])
        pltpu.sync_copy(rows.at[0], acc_out.at[pl.ds(c * NP + s * RPS + t * CH, CH)])


def _node_body(mx, acc, cnt, gs, w_out, b_out, wn1, wn2, wn3, nb_in,
               wres, bres, wno, bno, out):
    s_sum = acc[0] + acc[1]
    cvec = cnt[0, :, 0:1] + cnt[1, :, 0:1]
    so = jnp.dot(s_sum, w_out[...], preferred_element_type=jnp.float32)
    agg = jnp.where(cvec > 0.0, so / jnp.maximum(cvec, 1.0) + b_out[...], 0.0)
    h = (jnp.dot(mx[...], wn1[...], preferred_element_type=jnp.float32)
         + jnp.dot(agg, wn2[...], preferred_element_type=jnp.float32)
         + gs[...] * wn3[...] + nb_in[...])
    h = jnp.maximum(h, 0.0)
    h = jnp.maximum(
        h + jnp.dot(h, wres[...], preferred_element_type=jnp.float32)
        + bres[...], 0.0)
    out[...] = jnp.dot(h, wno[...], preferred_element_type=jnp.float32) + bno[...]


def _node_call(mx, acc, cnt, gs, w_out, b_out, wn1, wn2, wn3, nb_in,
               wres, bres, wno, bno):
    bn = 1000
    full = lambda i: (0, 0)
    return pl.pallas_call(
        _node_body,
        grid=(N // bn,),
        in_specs=[
            pl.BlockSpec((bn, D), lambda i: (i, 0)),
            pl.BlockSpec((NC, bn, H), lambda i: (0, i, 0)),
            pl.BlockSpec((NC, bn, H), lambda i: (0, i, 0)),
            pl.BlockSpec((bn, 1), lambda i: (i, 0)),
            pl.BlockSpec((H, D), full),
            pl.BlockSpec((1, D), full),
            pl.BlockSpec((D, H), full),
            pl.BlockSpec((D, H), full),
            pl.BlockSpec((1, H), full),
            pl.BlockSpec((1, H), full),
            pl.BlockSpec((H, H), full),
            pl.BlockSpec((1, H), full),
            pl.BlockSpec((H, D), full),
            pl.BlockSpec((1, D), full),
        ],
        out_specs=pl.BlockSpec((bn, D), lambda i: (i, 0)),
        out_shape=jax.ShapeDtypeStruct((N, D), jnp.float32),
    )(mx, acc, cnt, gs, w_out, b_out, wn1, wn2, wn3, nb_in, wres, bres,
      wno, bno)


def kernel(meta_xs, edge_index, edge_feature, global_state,
           bWin, bbin, bWres, bbres, bWout, bbout,
           nWin, nbin, nWres, nbres, nWout, nbout):
    ei3 = edge_index[0].reshape(NW, NCH, CH)
    ej3 = edge_index[1].reshape(NW, NCH, CH)
    w1, w2, w3 = bWin[:D], bWin[D:2 * D], bWin[2 * D:]
    p, q = _pq_call(meta_xs, w1, w2)
    g = _gather_k(p, q, ei3, ej3)
    cnt_f = _count_k(ei3)
    h = _edge_call(g, edge_feature, w3, bbin.reshape(1, H),
                   bWres, bbres.reshape(1, H))
    acc = _scatter_k(h, ei3).reshape(NC, NP, H)
    cnt = cnt_f.reshape(NC, NP, H)
    wn1, wn2, wn3 = nWin[:D], nWin[D:2 * D], nWin[2 * D:2 * D + 1]
    return _node_call(
        meta_xs, acc, cnt, global_state.reshape(N, 1),
        bWout, bbout.reshape(1, D), wn1, wn2, wn3, nbin.reshape(1, H),
        nWres, nbres.reshape(1, H), nWout, nbout.reshape(1, D))
